# baseline (speedup 1.0000x reference)
import jax
import jax.numpy as jnp
from jax.experimental import pallas as pl
from jax.experimental.pallas import tpu as pltpu

NUM_EMB = 5
EMB_DIM = 32
UNITS = 128
BURN = 10
SEQ = 50


def _lstm_kernel(ids_ref, table_ref, w_ref, r_ref, b_ref, out_ref,
                 hs_ref, dma_sem):
    B = ids_ref.shape[0]
    proj = (
        jax.lax.dot_general(
            table_ref[:], w_ref[:], (((1,), (0,)), ((), ())),
            preferred_element_type=jnp.float32,
        )
        + b_ref[:]
    )
    lane4 = jax.lax.broadcasted_iota(jnp.int32, (1, 4 * UNITS), 1)
    gate_scale = jnp.where(
        (lane4 >= 2 * UNITS) & (lane4 < 3 * UNITS), 1.0, 0.5
    ).astype(jnp.float32)
    proj = proj * gate_scale
    rec = r_ref[:] * gate_scale
    NP = 1
    PB = B // NP
    ids_val = ids_ref[:]

    def substep(t, p, h, c):
        idx_col = pltpu.roll(ids_val, -t, axis=1)[
            p * PB:(p + 1) * PB, 0:1]
        idxb = jnp.broadcast_to(idx_col, (PB, 4 * UNITS))
        zx = jnp.take_along_axis(proj, idxb, axis=0)
        z = zx + jax.lax.dot_general(
            h, rec, (((1,), (0,)), ((), ())),
            preferred_element_type=jnp.float32,
        )
        ti = jnp.tanh(z[:, :UNITS])
        tf = jnp.tanh(z[:, UNITS:2 * UNITS])
        g = jnp.tanh(z[:, 2 * UNITS:3 * UNITS])
        to = jnp.tanh(z[:, 3 * UNITS:])
        c_new = 0.5 * ((tf * c + c) + (ti * g + g))
        tc = jnp.tanh(c_new)
        h_new = 0.5 * (to * tc + tc)

        @pl.when(t >= BURN)
        def _():
            hs_ref[pl.ds(t - BURN, 1), p * PB:(p + 1) * PB, :] = (
                h_new[None, :, :]
            )
            pltpu.make_async_copy(
                hs_ref.at[t - BURN], out_ref.at[:, t - BURN], dma_sem
            ).start()

        return h_new, c_new

    def step(t, carry):
        return tuple(substep(t, p, h, c) for p, (h, c) in enumerate(carry))

    zero = jnp.zeros((PB, UNITS), dtype=jnp.float32)
    jax.lax.fori_loop(0, SEQ, step, tuple((zero, zero) for _ in range(NP)),
                      unroll=10)

    def drain(k, _):
        pltpu.make_async_copy(
            hs_ref.at[k], out_ref.at[:, k], dma_sem
        ).wait()
        return 0

    jax.lax.fori_loop(0, SEQ - BURN, drain, 0)


def kernel(weekday_ids, table, kernel, rec_kernel, bias):
    B, L = weekday_ids.shape
    bias2d = bias.reshape(1, -1)
    return pl.pallas_call(
        _lstm_kernel,
        out_shape=jax.ShapeDtypeStruct((B, L - BURN, UNITS), jnp.float32),
        out_specs=pl.BlockSpec(memory_space=pl.ANY),
        scratch_shapes=[
            pltpu.VMEM((L - BURN, B, UNITS), jnp.float32),
            pltpu.SemaphoreType.DMA,
        ],
    )(weekday_ids, table, kernel, rec_kernel, bias2d)


# speedup vs baseline: 1.2724x; 1.2724x over previous
import jax
import jax.numpy as jnp
from jax.experimental import pallas as pl
from jax.experimental.pallas import tpu as pltpu

NUM_EMB = 5
EMB_DIM = 32
UNITS = 128
BURN = 10
SEQ = 50


def _lstm_kernel(ids_ref, table_ref, w_ref, r_ref, b_ref, out_ref,
                 hs_ref, dma_sem):
    B = ids_ref.shape[1]
    proj = (
        jax.lax.dot_general(
            table_ref[:], w_ref[:], (((1,), (0,)), ((), ())),
            preferred_element_type=jnp.float32,
        )
        + b_ref[:]
    )
    lane4 = jax.lax.broadcasted_iota(jnp.int32, (1, 4 * UNITS), 1)
    gate_scale = jnp.where(
        (lane4 >= 2 * UNITS) & (lane4 < 3 * UNITS), 1.0, 0.5
    ).astype(jnp.float32)
    proj = proj * gate_scale
    rec = r_ref[:] * gate_scale

    iota_e = jax.lax.broadcasted_iota(jnp.int32, (NUM_EMB, B), 0)
    wcat = jnp.concatenate([rec, proj], axis=0)

    def step(t, carry):
        hT, cT = carry
        ids_t = ids_ref[pl.ds(t, 1), :]
        onehot_t = (ids_t == iota_e).astype(jnp.float32)
        xcat = jnp.concatenate([hT, onehot_t], axis=0)
        zT = jax.lax.dot_general(
            wcat, xcat, (((0,), (0,)), ((), ())),
            preferred_element_type=jnp.float32,
        )
        ti = jnp.tanh(zT[:UNITS, :])
        tf = jnp.tanh(zT[UNITS:2 * UNITS, :])
        g = jnp.tanh(zT[2 * UNITS:3 * UNITS, :])
        to = jnp.tanh(zT[3 * UNITS:, :])
        c_new = 0.5 * ((tf * cT + cT) + (ti * g + g))
        tc = jnp.tanh(c_new)
        h_new = 0.5 * (to * tc + tc)

        @pl.when(t >= BURN)
        def _():
            hs_ref[pl.ds(t - BURN, 1), :, :] = (
                jnp.swapaxes(h_new, 0, 1)[None, :, :]
            )
            pltpu.make_async_copy(
                hs_ref.at[t - BURN], out_ref.at[:, t - BURN], dma_sem
            ).start()

        return (h_new, c_new)

    zero = jnp.zeros((UNITS, B), dtype=jnp.float32)
    jax.lax.fori_loop(0, SEQ, step, (zero, zero), unroll=10)

    def drain(k, _):
        pltpu.make_async_copy(
            hs_ref.at[k], out_ref.at[:, k], dma_sem
        ).wait()
        return 0

    jax.lax.fori_loop(0, SEQ - BURN, drain, 0)


def kernel(weekday_ids, table, kernel, rec_kernel, bias):
    B, L = weekday_ids.shape
    ids_t = weekday_ids.T
    bias2d = bias.reshape(1, -1)
    return pl.pallas_call(
        _lstm_kernel,
        out_shape=jax.ShapeDtypeStruct((B, L - BURN, UNITS), jnp.float32),
        out_specs=pl.BlockSpec(memory_space=pl.ANY),
        scratch_shapes=[
            pltpu.VMEM((L - BURN, B, UNITS), jnp.float32),
            pltpu.SemaphoreType.DMA,
        ],
    )(ids_t, table, kernel, rec_kernel, bias2d)


# speedup vs baseline: 1.2800x; 1.0060x over previous
import jax
import jax.numpy as jnp
from jax.experimental import pallas as pl
from jax.experimental.pallas import tpu as pltpu

NUM_EMB = 5
EMB_DIM = 32
UNITS = 128
BURN = 10
SEQ = 50


def _lstm_kernel(ids_ref, table_ref, w_ref, r_ref, b_ref, out_ref,
                 hs_ref, dma_sem):
    B = ids_ref.shape[1]
    proj = (
        jax.lax.dot_general(
            table_ref[:], w_ref[:], (((1,), (0,)), ((), ())),
            preferred_element_type=jnp.float32,
        )
        + b_ref[:]
    )
    lane4 = jax.lax.broadcasted_iota(jnp.int32, (1, 4 * UNITS), 1)
    gate_scale = jnp.where(
        (lane4 >= 2 * UNITS) & (lane4 < 3 * UNITS), 1.0, 0.5
    ).astype(jnp.float32)
    proj = proj * gate_scale
    rec = r_ref[:] * gate_scale

    iota_e = jax.lax.broadcasted_iota(jnp.int32, (NUM_EMB, B), 0)
    wcat = jnp.concatenate([rec, proj], axis=0)

    def step(t, carry):
        hT, cT = carry
        ids_t = ids_ref[pl.ds(t, 1), :]
        onehot_t = (ids_t == iota_e).astype(jnp.float32)
        xcat = jnp.concatenate([hT, onehot_t], axis=0)
        zT = jax.lax.dot_general(
            wcat, xcat, (((0,), (0,)), ((), ())),
            preferred_element_type=jnp.float32,
        )
        ti = jnp.tanh(zT[:UNITS, :])
        tf = jnp.tanh(zT[UNITS:2 * UNITS, :])
        g = jnp.tanh(zT[2 * UNITS:3 * UNITS, :])
        to = jnp.tanh(zT[3 * UNITS:, :])
        c_new = 0.5 * ((tf * cT + cT) + (ti * g + g))
        tc = jnp.tanh(c_new)
        h_new = 0.5 * (to * tc + tc)

        @pl.when(t >= BURN)
        def _():
            hs_ref[pl.ds(t - BURN, 1), :, :] = (
                jnp.swapaxes(h_new, 0, 1)[None, :, :]
            )
            pltpu.make_async_copy(
                hs_ref.at[t - BURN], out_ref.at[:, t - BURN], dma_sem
            ).start()

        return (h_new, c_new)

    zero = jnp.zeros((UNITS, B), dtype=jnp.float32)
    jax.lax.fori_loop(0, SEQ, step, (zero, zero), unroll=25)

    def drain(k, _):
        pltpu.make_async_copy(
            hs_ref.at[k], out_ref.at[:, k], dma_sem
        ).wait()
        return 0

    jax.lax.fori_loop(0, SEQ - BURN, drain, 0)


def kernel(weekday_ids, table, kernel, rec_kernel, bias):
    B, L = weekday_ids.shape
    ids_t = weekday_ids.T
    bias2d = bias.reshape(1, -1)
    return pl.pallas_call(
        _lstm_kernel,
        out_shape=jax.ShapeDtypeStruct((B, L - BURN, UNITS), jnp.float32),
        out_specs=pl.BlockSpec(memory_space=pl.ANY),
        scratch_shapes=[
            pltpu.VMEM((L - BURN, B, UNITS), jnp.float32),
            pltpu.SemaphoreType.DMA,
        ],
    )(ids_t, table, kernel, rec_kernel, bias2d)
